# baseline (speedup 1.0000x reference)
import functools

import jax
import jax.numpy as jnp
from jax import lax
from jax.experimental import pallas as pl
from jax.experimental.pallas import tpu as pltpu

_EPS = 1e-5
_V = 25
_TB = 8
_Q = _TB * _V
_PREC = lax.Precision.DEFAULT
_PREC_G = lax.Precision.HIGHEST


def _bnfold(g, b, m, v):
    s = g / jnp.sqrt(v + _EPS)
    return s, b - s * m


def _g_kernel(x_ref, w1_ref, b1_ref, w2_ref, b2_ref, m_ref, g_ref, *, tc):
    w1 = w1_ref[...]
    b1 = b1_ref[...]
    w2 = w2_ref[...]
    b2 = b2_ref[...]
    mask = m_ref[...]
    for k in range(tc):
        x = x_ref[k]
        p = jnp.dot(w1, x, preferred_element_type=jnp.float32,
                    precision=_PREC_G) + b1
        q = jnp.dot(w2, x, preferred_element_type=jnp.float32,
                    precision=_PREC_G) + b2
        s = lax.dot_general(p, q, (((0,), (0,)), ((), ())),
                            preferred_element_type=jnp.float32,
                            precision=_PREC_G) + mask
        s = s - jnp.max(s, axis=-1, keepdims=True)
        e = jnp.exp(s)
        g_ref[k] = (e / jnp.sum(e, axis=-1, keepdims=True)).astype(g_ref.dtype)


def _compute_g(xq, wg1, bg1, wg2, bg2):
    n, tc, cin, q = xq.shape
    dg = wg1.shape[0]
    aq = jnp.arange(q) // _V
    mask = jnp.where(aq[:, None] == aq[None, :], 0.0, -1e30).astype(jnp.float32)
    return pl.pallas_call(
        functools.partial(_g_kernel, tc=tc),
        out_shape=jax.ShapeDtypeStruct((n, tc, q, q), xq.dtype),
        grid=(n,),
        in_specs=[
            pl.BlockSpec((None, tc, cin, q), lambda i: (i, 0, 0, 0)),
            pl.BlockSpec((dg, cin), lambda i: (0, 0)),
            pl.BlockSpec((dg, 1), lambda i: (0, 0)),
            pl.BlockSpec((dg, cin), lambda i: (0, 0)),
            pl.BlockSpec((dg, 1), lambda i: (0, 0)),
            pl.BlockSpec((q, q), lambda i: (0, 0)),
        ],
        out_specs=pl.BlockSpec((None, tc, q, q), lambda i: (i, 0, 0, 0)),
        compiler_params=pltpu.CompilerParams(
            dimension_semantics=("parallel",)),
    )(xq, wg1, bg1.reshape(dg, 1), wg2, bg2.reshape(dg, 1), mask)


def _shear(z, sign, nrow):
    z3 = z.reshape(nrow, _TB, _V)
    ir = lax.broadcasted_iota(jnp.int32, (nrow, _TB, _V), 0)
    iv = lax.broadcasted_iota(jnp.int32, (nrow, _TB, _V), 2)
    idx = jnp.mod(iv + sign * ir, _V)
    return jnp.take_along_axis(z3, idx, axis=-1)


def _layer_chunk(x, res, g_ref, k, lwT, mk, s1, bb, ww, ww1, cc, o_ref, c, d):
    xs3 = _shear(x, 1, c) * mk[:, None, :]
    t1 = jnp.dot(lwT, xs3.reshape(c, _Q),
                 preferred_element_type=jnp.float32, precision=_PREC)
    ys3 = _shear(t1, -1, d) * s1[:, None, :] + bb[:, None, :]
    h = jnp.maximum(ys3.reshape(d, _Q) + res, 0.0)
    a = lax.dot_general(h, g_ref[k], (((1,), (1,)), ((), ())),
                        preferred_element_type=jnp.float32, precision=_PREC)
    out = (jnp.dot(ww, a, preferred_element_type=jnp.float32, precision=_PREC)
           + jnp.dot(ww1, h, preferred_element_type=jnp.float32,
                     precision=_PREC)
           + cc)
    o_ref[k] = jnp.maximum(out, 0.0).astype(o_ref.dtype)


def _layer_kernel(x_ref, g_ref, lw_ref, mk_ref, s1_ref, bb_ref,
                  ww_ref, ww1_ref, cc_ref, o_ref, *, tc, c, d):
    lwT = lw_ref[...]
    mk = mk_ref[...]
    s1 = s1_ref[...]
    bb = bb_ref[...]
    ww = ww_ref[...]
    ww1 = ww1_ref[...]
    cc = cc_ref[...]
    for k in range(tc):
        x = x_ref[k].astype(jnp.float32)
        _layer_chunk(x, x, g_ref, k, lwT, mk, s1, bb, ww, ww1, cc, o_ref, c, d)


def _layer_down_kernel(x_ref, g_ref, lw_ref, mk_ref, s1_ref, bb_ref,
                       wd_ref, cd_ref, ww_ref, ww1_ref, cc_ref, o_ref,
                       *, tc, c, d):
    lwT = lw_ref[...]
    mk = mk_ref[...]
    s1 = s1_ref[...]
    bb = bb_ref[...]
    wd = wd_ref[...]
    cd = cd_ref[...]
    ww = ww_ref[...]
    ww1 = ww1_ref[...]
    cc = cc_ref[...]
    for k in range(tc):
        x = x_ref[k].astype(jnp.float32)
        res = jnp.dot(wd, x, preferred_element_type=jnp.float32,
                      precision=_PREC) + cd
        _layer_chunk(x, res, g_ref, k, lwT, mk, s1, bb, ww, ww1, cc, o_ref,
                     c, d)


def _gcn_layer(x0q, g, Lw, Lb, FM, bn1, Ww, Ww1, bw1, bns, down):
    n, tc, c, q = x0q.shape
    d = Lw.shape[1]
    mask_cv = jnp.tanh(FM[0]).T + 1.0
    s1, b1 = _bnfold(*bn1)
    s1_dv = s1.reshape(_V, d).T
    b1_dv = b1.reshape(_V, d).T
    bb = Lb[:, None] * s1_dv + b1_dv
    ss, bs = _bnfold(*bns)
    ww = Ww * ss[:, None]
    ww1 = Ww1 * ss[:, None]
    cc = (ss * bw1 + bs).reshape(d, 1)
    f32 = jnp.float32
    in_specs = [
        pl.BlockSpec((None, tc, c, q), lambda i: (i, 0, 0, 0)),
        pl.BlockSpec((None, tc, q, q), lambda i: (i, 0, 0, 0)),
        pl.BlockSpec((d, c), lambda i: (0, 0)),
        pl.BlockSpec((c, _V), lambda i: (0, 0)),
        pl.BlockSpec((d, _V), lambda i: (0, 0)),
        pl.BlockSpec((d, _V), lambda i: (0, 0)),
    ]
    args = [x0q, g, jnp.transpose(Lw).astype(f32), mask_cv.astype(f32),
            s1_dv.astype(f32), bb.astype(f32)]
    if down is None:
        kern = functools.partial(_layer_kernel, tc=tc, c=c, d=d)
    else:
        dw, db, dbn = down
        sd, bd = _bnfold(*dbn)
        kern = functools.partial(_layer_down_kernel, tc=tc, c=c, d=d)
        in_specs += [
            pl.BlockSpec((d, c), lambda i: (0, 0)),
            pl.BlockSpec((d, 1), lambda i: (0, 0)),
        ]
        args += [(dw * sd[:, None]).astype(f32),
                 (sd * db + bd).reshape(d, 1).astype(f32)]
    in_specs += [
        pl.BlockSpec((d, d), lambda i: (0, 0)),
        pl.BlockSpec((d, d), lambda i: (0, 0)),
        pl.BlockSpec((d, 1), lambda i: (0, 0)),
    ]
    args += [ww.astype(f32), ww1.astype(f32), cc.astype(f32)]
    return pl.pallas_call(
        kern,
        out_shape=jax.ShapeDtypeStruct((n, tc, d, q), x0q.dtype),
        grid=(n,),
        in_specs=in_specs,
        out_specs=pl.BlockSpec((None, tc, d, q), lambda i: (i, 0, 0, 0)),
        compiler_params=pltpu.CompilerParams(
            dimension_semantics=("parallel",)),
    )(*args)


def _tcn_kernel(h_ref, x_ref, wt_ref, wr_ref, c_ref, o_ref, *, cout, taps):
    hf = h_ref[...]
    acc = jnp.dot(wr_ref[...], x_ref[...],
                  preferred_element_type=jnp.float32, precision=_PREC)
    for k in range(taps):
        s = (k - (taps - 1) // 2) * _V
        if s > 0:
            xk = jnp.concatenate(
                [hf[:, s:], jnp.zeros((cout, s), hf.dtype)], axis=1)
        elif s < 0:
            xk = jnp.concatenate(
                [jnp.zeros((cout, -s), hf.dtype), hf[:, :s]], axis=1)
        else:
            xk = hf
        acc = acc + jnp.dot(wt_ref[k], xk,
                            preferred_element_type=jnp.float32,
                            precision=_PREC)
    acc = acc + c_ref[...]
    o_ref[...] = jnp.maximum(acc, 0.0).astype(o_ref.dtype)


def _tcn(hf, xf, wt, wr, ctot):
    n, cout, m = hf.shape
    cin = xf.shape[1]
    taps = wt.shape[0]
    kern = functools.partial(_tcn_kernel, cout=cout, taps=taps)
    return pl.pallas_call(
        kern,
        out_shape=jax.ShapeDtypeStruct((n, cout, m), hf.dtype),
        grid=(n,),
        in_specs=[
            pl.BlockSpec((None, cout, m), lambda i: (i, 0, 0)),
            pl.BlockSpec((None, cin, m), lambda i: (i, 0, 0)),
            pl.BlockSpec((taps, cout, cout), lambda i: (0, 0, 0)),
            pl.BlockSpec((cout, cin), lambda i: (0, 0)),
            pl.BlockSpec((cout, 1), lambda i: (0, 0)),
        ],
        out_specs=pl.BlockSpec((None, cout, m), lambda i: (i, 0, 0)),
        compiler_params=pltpu.CompilerParams(
            dimension_semantics=("parallel",)),
    )(hf, xf, wt.astype(jnp.float32), wr.astype(jnp.float32),
      ctot.reshape(cout, 1).astype(jnp.float32))


def kernel(x, g1_w, g1_b, g2_w, g2_b,
           l1_Lw, l1_Lb, l1_FM, l1_bn1_g, l1_bn1_b, l1_bn1_m, l1_bn1_v,
           l1_Ww, l1_Ww1, l1_bw1, l1_bns_g, l1_bns_b, l1_bns_m, l1_bns_v,
           l2_Lw, l2_Lb, l2_FM, l2_bn1_g, l2_bn1_b, l2_bn1_m, l2_bn1_v,
           l2_Ww, l2_Ww1, l2_bw1, l2_bns_g, l2_bns_b, l2_bns_m, l2_bns_v,
           l2_dw, l2_db, l2_dbn_g, l2_dbn_b, l2_dbn_m, l2_dbn_v,
           l3_Lw, l3_Lb, l3_FM, l3_bn1_g, l3_bn1_b, l3_bn1_m, l3_bn1_v,
           l3_Ww, l3_Ww1, l3_bw1, l3_bns_g, l3_bns_b, l3_bns_m, l3_bns_v,
           t_w, t_b, t_bn_g, t_bn_b, t_bn_m, t_bn_v,
           r_w, r_b, r_bn_g, r_bn_b, r_bn_m, r_bn_v):
    n, c, t, v = x.shape
    m = t * v
    tc = t // _TB
    xq = jnp.swapaxes(x.reshape(n, c, tc, _Q), 1, 2)

    g = _compute_g(xq, g1_w, g1_b, g2_w, g2_b)

    h = _gcn_layer(xq, g, l1_Lw, l1_Lb, l1_FM,
                   (l1_bn1_g, l1_bn1_b, l1_bn1_m, l1_bn1_v),
                   l1_Ww, l1_Ww1, l1_bw1,
                   (l1_bns_g, l1_bns_b, l1_bns_m, l1_bns_v), None)
    h = _gcn_layer(h, g, l2_Lw, l2_Lb, l2_FM,
                   (l2_bn1_g, l2_bn1_b, l2_bn1_m, l2_bn1_v),
                   l2_Ww, l2_Ww1, l2_bw1,
                   (l2_bns_g, l2_bns_b, l2_bns_m, l2_bns_v),
                   (l2_dw, l2_db, (l2_dbn_g, l2_dbn_b, l2_dbn_m, l2_dbn_v)))
    h = _gcn_layer(h, g, l3_Lw, l3_Lb, l3_FM,
                   (l3_bn1_g, l3_bn1_b, l3_bn1_m, l3_bn1_v),
                   l3_Ww, l3_Ww1, l3_bw1,
                   (l3_bns_g, l3_bns_b, l3_bns_m, l3_bns_v), None)

    cout = h.shape[2]
    hf = jnp.swapaxes(h, 1, 2).reshape(n, cout, m)
    sr, br = _bnfold(r_bn_g, r_bn_b, r_bn_m, r_bn_v)
    wr = r_w[:, :, 0] * sr[:, None]
    cr = sr * r_b + br
    st, bt = _bnfold(t_bn_g, t_bn_b, t_bn_m, t_bn_v)
    wt = jnp.transpose(t_w, (2, 0, 1)) * st[None, :, None]
    ctot = st * t_b + bt + cr
    out = _tcn(hf, x.reshape(n, c, m), wt, wr, ctot)
    return out.reshape(n, cout, t, v)
